# baseline (speedup 1.0000x reference)
import functools

import jax
import jax.numpy as jnp
from jax import lax
from jax.experimental import pallas as pl
from jax.experimental.pallas import tpu as pltpu
from jax.experimental.pallas import tpu_sc as plsc

D_MODEL = 64

_info = plsc.get_sparse_core_info()
_NC, _NS, _L = _info.num_cores, _info.num_subcores, _info.num_lanes
_NW = _NC * _NS

_IB = 256
_NBUF = 2


def _emb_body(x_hbm, table_hbm, out_hbm, xstage, idxb, gb, stage, sems):
    n_batch = x_hbm.shape[0]
    n_seq = x_hbm.shape[1]
    b_per_w = n_batch // _NW
    nsub = b_per_w // _IB
    n_steps = n_seq * nsub
    wid = lax.axis_index("s") * _NC + lax.axis_index("c")
    i0 = wid * b_per_w

    sem_g = (sems[0], sems[1])
    sem_s = (sems[2], sems[3])
    iota = lax.iota(jnp.int32, _L)

    pltpu.sync_copy(x_hbm.at[pl.ds(i0, b_per_w), :], xstage)

    def prep_idx(t, b):
        j = t // nsub
        ioff = (t % nsub) * _IB
        jvec = jnp.zeros((_L,), jnp.int32) + j

        @pl.loop(0, _IB // _L)
        def _g(ig):
            rows = ioff + ig * _L + iota
            idxb[b][pl.ds(ig * _L, _L)] = plsc.load_gather(xstage, [rows, jvec])

    def fire_gather(b):
        pltpu.async_copy(table_hbm.at[idxb[b]], gb[b], sem_g[b])

    def wait_gather(b):
        pltpu.make_async_copy(table_hbm.at[pl.ds(0, _IB)], gb[b], sem_g[b]).wait()

    def extract(b):
        @pl.loop(0, _IB // _L)
        def _g(ig):
            rows = ig * _L + iota
            for k in range(D_MODEL):
                kvec = jnp.full((_L,), k, jnp.int32)
                stage[b][k, pl.ds(ig * _L, _L)] = plsc.load_gather(
                    gb[b], [rows, kvec]
                )

    def out_slice(t):
        j = t // nsub
        ioff = i0 + (t % nsub) * _IB
        return out_hbm.at[j, :, pl.ds(ioff, _IB)]

    def fire_store(t, b):
        pltpu.async_copy(stage[b], out_slice(t), sem_s[b])

    def wait_store(t, b):
        pltpu.make_async_copy(stage[b], out_slice(t), sem_s[b]).wait()

    prep_idx(0, 0)
    fire_gather(0)

    @pl.loop(0, n_steps, step=_NBUF)
    def _step(t0):
        for b in range(_NBUF):
            t = t0 + b

            @pl.when(t >= _NBUF)
            def _drain():
                wait_store(t - _NBUF, b)

            @pl.when(t + 1 < n_steps)
            def _pref():
                prep_idx(t + 1, 1 - b)
                fire_gather(1 - b)

            wait_gather(b)
            extract(b)
            fire_store(t, b)

    for b in range(_NBUF):
        wait_store(n_steps - _NBUF + b, b)


@jax.jit
def _emb_lookup(x2d, weight):
    n_batch, n_seq = x2d.shape
    outp = pl.kernel(
        _emb_body,
        out_type=jax.ShapeDtypeStruct((n_seq, D_MODEL, n_batch), jnp.float32),
        mesh=plsc.VectorSubcoreMesh(core_axis_name="c", subcore_axis_name="s"),
        scratch_types=[
            pltpu.VMEM((n_batch // _NW, n_seq), jnp.int32),
            [pltpu.VMEM((_IB,), jnp.int32) for _ in range(_NBUF)],
            [pltpu.VMEM((_IB, D_MODEL), jnp.float32) for _ in range(_NBUF)],
            [pltpu.VMEM((D_MODEL, _IB), jnp.float32) for _ in range(_NBUF)],
            [pltpu.SemaphoreType.DMA for _ in range(4)],
        ],
        compiler_params=pltpu.CompilerParams(use_tc_tiling_on_sc=False, needs_layout_passes=False),
    )(x2d, weight)
    return jnp.transpose(outp, (2, 0, 1))


def kernel(x, weight):
    return _emb_lookup(x, weight)


# speedup vs baseline: 1.6374x; 1.6374x over previous
import jax
import jax.numpy as jnp
from jax import lax
from jax.experimental import pallas as pl
from jax.experimental.pallas import tpu as pltpu
from jax.experimental.pallas import tpu_sc as plsc

D_MODEL = 64

_info = plsc.get_sparse_core_info()
_NC, _NS = _info.num_cores, _info.num_subcores
_NW = _NC * _NS

_IDX_W = 128
_CHUNK = 512
_K = _CHUNK // _IDX_W
_NBUF = 2


def _emb_body(x_hbm, table_hbm, out_hbm, idx_all, rows, sg0, sg1, ss0, ss1):
    sem_g = (sg0, sg1)
    sem_s = (ss0, ss1)
    wid = lax.axis_index("s") * _NC + lax.axis_index("c")
    n_rows_total = out_hbm.shape[0]
    b_per_w = n_rows_total // _NW
    n_chunks = b_per_w // _CHUNK
    idx_rows = b_per_w // _IDX_W
    base = wid * b_per_w

    pltpu.sync_copy(
        x_hbm.at[pl.ds(pl.multiple_of(base, 8), b_per_w)], idx_all
    )

    def buf(b):
        return rows.at[pl.ds(b * _CHUNK, _CHUNK)]

    def fire_gather(ci, b):
        pltpu.async_copy(
            table_hbm.at[idx_all.at[pl.ds(ci * _CHUNK, _CHUNK)]],
            buf(b),
            sem_g[b],
        )

    def wait_gather(b):
        pltpu.make_async_copy(table_hbm.at[pl.ds(0, _CHUNK)], buf(b), sem_g[b]).wait()

    def out_slice(ci):
        return out_hbm.at[pl.ds(pl.multiple_of(base + ci * _CHUNK, _CHUNK), _CHUNK)]

    def wait_store(ci, b):
        pltpu.make_async_copy(buf(b), out_slice(ci), sem_s[b]).wait()

    fire_gather(0, 0)
    fire_gather(1, 1)

    @pl.loop(0, n_chunks, step=_NBUF)
    def _chunk(ci0):
        for b in range(_NBUF):
            ci = ci0 + b
            wait_gather(b)
            pltpu.async_copy(buf(b), out_slice(ci), sem_s[b])

            @pl.when(ci < n_chunks - _NBUF)
            def _prefetch():
                wait_store(ci, b)
                fire_gather(ci + _NBUF, b)

    for b in range(_NBUF):
        wait_store(n_chunks - _NBUF + b, b)


@jax.jit
def _emb_lookup(x2d, weight):
    n = x2d.shape[0] * x2d.shape[1]
    out = pl.kernel(
        _emb_body,
        out_type=jax.ShapeDtypeStruct((n, D_MODEL), jnp.float32),
        mesh=plsc.VectorSubcoreMesh(core_axis_name="c", subcore_axis_name="s"),
        scratch_types=[
            pltpu.VMEM((n // _NW,), jnp.int32),
            pltpu.VMEM((_NBUF * _CHUNK, D_MODEL), jnp.float32),
            pltpu.SemaphoreType.DMA,
            pltpu.SemaphoreType.DMA,
            pltpu.SemaphoreType.DMA,
            pltpu.SemaphoreType.DMA,
        ],
        compiler_params=pltpu.CompilerParams(use_tc_tiling_on_sc=False),
    )(x2d.reshape(n), weight)
    return out.reshape(x2d.shape[0], x2d.shape[1], D_MODEL)


def kernel(x, weight):
    return _emb_lookup(x, weight)
